# Initial kernel scaffold; baseline (speedup 1.0000x reference)
import jax
import jax.numpy as jnp
from jax.experimental import pallas as pl


def kernel(pts, rand_vals, idx):
    raise NotImplementedError("write your pallas kernel here")


# speedup vs baseline: 1.7924x; 1.7924x over previous
import functools

import jax
import jax.numpy as jnp
from jax import lax
from jax.experimental import pallas as pl
from jax.experimental.pallas import tpu as pltpu
from jax.experimental.pallas import tpu_sc as plsc

_NUM_POINTS = 1000000
_PT_DIM = 64
_N_REP = 16384
_NC, _NS = 2, 16
_NW = _NC * _NS
_ROWS_PER_W = _N_REP // _NW
_CHUNK = 128
_CHUNKS_PER_W = _ROWS_PER_W // _CHUNK


@functools.cache
def _make_scatter():
    mesh = plsc.VectorSubcoreMesh(
        core_axis_name="c", subcore_axis_name="s", num_cores=_NC, num_subcores=_NS
    )

    @functools.partial(
        pl.kernel,
        mesh=mesh,
        compiler_params=pltpu.CompilerParams(use_tc_tiling_on_sc=False),
        scratch_types=[
            pltpu.VMEM((_CHUNKS_PER_W, _CHUNK), jnp.int32),
            pltpu.VMEM((_ROWS_PER_W, _PT_DIM), jnp.float32),
            pltpu.SemaphoreType.DMA,
        ],
    )
    def _scatter_rows(pts_ref, rv_hbm, idx_hbm, idx_v, rows_v, sem):
        w = lax.axis_index("s") * _NC + lax.axis_index("c")
        pltpu.sync_copy(idx_hbm.at[pl.ds(w * _CHUNKS_PER_W, _CHUNKS_PER_W)], idx_v)
        pltpu.sync_copy(rv_hbm.at[pl.ds(w * _ROWS_PER_W, _ROWS_PER_W)], rows_v)
        copies = [
            pltpu.async_copy(
                rows_v.at[pl.ds(j * _CHUNK, _CHUNK)],
                pts_ref.at[idx_v.at[j]],
                sem,
            )
            for j in range(_CHUNKS_PER_W)
        ]
        for c in copies:
            c.wait()

    return _scatter_rows


def kernel(pts, rand_vals, idx):
    idx2 = idx.astype(jnp.int32).reshape(_NW * _CHUNKS_PER_W, _CHUNK)
    pts_ref = jax.new_ref(pts)
    _make_scatter()(pts_ref, rand_vals, idx2)
    return pts_ref[...]


# speedup vs baseline: 1.7955x; 1.0017x over previous
import functools

import jax
import jax.numpy as jnp
from jax import lax
from jax.experimental import pallas as pl
from jax.experimental.pallas import tpu as pltpu
from jax.experimental.pallas import tpu_sc as plsc

_NUM_POINTS = 1000000
_PT_DIM = 64
_N_REP = 16384
_NC, _NS = 2, 16
_NW = _NC * _NS
_ROWS_PER_W = _N_REP // _NW
_CHUNK = 128
_CHUNKS_PER_W = _ROWS_PER_W // _CHUNK


@functools.cache
def _make_scatter():
    mesh = plsc.VectorSubcoreMesh(
        core_axis_name="c", subcore_axis_name="s", num_cores=_NC, num_subcores=_NS
    )

    @functools.partial(
        pl.kernel,
        mesh=mesh,
        compiler_params=pltpu.CompilerParams(use_tc_tiling_on_sc=False),
        scratch_types=[
            pltpu.VMEM((_CHUNKS_PER_W, _CHUNK), jnp.int32),
            pltpu.VMEM((_ROWS_PER_W, _PT_DIM), jnp.float32),
            pltpu.SemaphoreType.DMA,
        ],
    )
    def _scatter_rows(pts_ref, rv_hbm, idx_hbm, idx_v, rows_v, sem):
        w = lax.axis_index("s") * _NC + lax.axis_index("c")
        pltpu.sync_copy(idx_hbm.at[pl.ds(w * _CHUNKS_PER_W, _CHUNKS_PER_W)], idx_v)
        pltpu.sync_copy(rv_hbm.at[pl.ds(w * _ROWS_PER_W, _ROWS_PER_W)], rows_v)
        copies = [
            pltpu.async_copy(
                rows_v.at[pl.ds(j * _CHUNK, _CHUNK)],
                pts_ref.at[idx_v.at[j]],
                sem,
            )
            for j in range(_CHUNKS_PER_W)
        ]
        for c in copies:
            c.wait()

    return _scatter_rows


def kernel(pts, rand_vals, idx):
    idx2 = idx.astype(jnp.int32).reshape(_NW * _CHUNKS_PER_W, _CHUNK)
    pts_ref = jax.new_ref(pts)
    _make_scatter()(pts_ref, rand_vals, idx2)
    return jax.freeze(pts_ref)
